# baseline (speedup 1.0000x reference)
import jax
import jax.numpy as jnp
from jax import lax
from jax.experimental import pallas as pl
from jax.experimental.pallas import tpu as pltpu
from jax.experimental.pallas import tpu_sc as plsc

_VOCAB = 1000000
_EMBED = 64
_B, _L = 1024, 200
_NW = 32
_ROWS = _B * _L
_RPW = _ROWS // _NW
_CHUNK = 128
_NCHUNK = _RPW // _CHUNK


def _embed_kernel(x_hbm, tok_hbm, pos_hbm, out_hbm, idx_v, tok_buf, pos2, sem):
    c = lax.axis_index("c")
    s = lax.axis_index("s")
    wid = s * 2 + c
    pltpu.sync_copy(pos_hbm.at[pl.ds(0, _L)], pos2.at[pl.ds(0, _L)])
    pltpu.sync_copy(pos_hbm.at[pl.ds(0, _L)], pos2.at[pl.ds(_L, _L)])
    base0 = wid * _RPW

    def chunk_body(ci, carry):
        base = base0 + ci * _CHUNK
        q = lax.rem(ci * _CHUNK, _L)
        pltpu.sync_copy(x_hbm.at[pl.ds(base, _CHUNK)], idx_v)
        pltpu.async_copy(tok_hbm.at[idx_v], tok_buf, sem).wait()

        def row_body(r, carry2):
            pr = q + r
            for j in range(4):
                sl = pl.ds(j * 16, 16)
                tok_buf[r, sl] = tok_buf[r, sl] + pos2[pr, sl]
            return carry2

        lax.fori_loop(0, _CHUNK, row_body, 0)
        pltpu.sync_copy(tok_buf, out_hbm.at[pl.ds(base, _CHUNK)])
        return carry

    lax.fori_loop(0, _NCHUNK, chunk_body, 0)


@jax.jit
def _embed(xf, tok_table, pos_table):
    mesh = plsc.VectorSubcoreMesh(core_axis_name="c", subcore_axis_name="s")
    return pl.kernel(
        _embed_kernel,
        out_type=jax.ShapeDtypeStruct((_ROWS, _EMBED), jnp.float32),
        mesh=mesh,
        scratch_types=[
            pltpu.VMEM((_CHUNK,), jnp.int32),
            pltpu.VMEM((_CHUNK, _EMBED), jnp.float32),
            pltpu.VMEM((2 * _L, _EMBED), jnp.float32),
            pltpu.SemaphoreType.DMA,
        ],
        compiler_params=pltpu.CompilerParams(use_tc_tiling_on_sc=False),
    )(xf, tok_table, pos_table)


def kernel(x, tok_table, pos_table):
    xf = jnp.reshape(x, (_ROWS,)).astype(jnp.int32)
    out = _embed(xf, tok_table, pos_table)
    return jnp.reshape(out, (_B, _L, _EMBED))


# speedup vs baseline: 1.2317x; 1.2317x over previous
import jax
import jax.numpy as jnp
from jax import lax
from jax.experimental import pallas as pl
from jax.experimental.pallas import tpu as pltpu
from jax.experimental.pallas import tpu_sc as plsc

_VOCAB = 1000000
_EMBED = 64
_B, _L = 1024, 200
_NW = 32
_ROWS = _B * _L
_RPW = _ROWS // _NW
_SUPER = 2 * _L
_NSUPER = _RPW // _SUPER
_G = 100
_NG = _SUPER // _G


def _embed_kernel(x_hbm, tok_hbm, pos_hbm, out_hbm,
                  idx_all, tok0, tok1, pos_v,
                  gsem0, gsem1, wsem0, wsem1):
    c = lax.axis_index("c")
    s = lax.axis_index("s")
    wid = s * 2 + c
    bufs = (tok0, tok1)
    gsems = (gsem0, gsem1)
    wsems = (wsem0, wsem1)

    pltpu.sync_copy(pos_hbm.at[pl.ds(0, _L)], pos_v)
    pltpu.sync_copy(x_hbm.at[pl.ds(wid * (_RPW // _G), _RPW // _G)], idx_all)

    descs = {}

    def start(sc):
        b = sc & 1
        if sc >= 2:
            pltpu.make_async_copy(out_hbm.at[pl.ds(0, _SUPER)],
                                  bufs[b], wsems[b]).wait()
        dlist = []
        for j in range(_NG):
            d = pltpu.async_copy(
                tok_hbm.at[idx_all.at[_NG * sc + j]],
                bufs[b].at[pl.ds(_G * j, _G)],
                gsems[b])
            dlist.append(d)
        descs[sc] = dlist

    def process(sc):
        b = sc & 1
        for d in descs[sc]:
            d.wait()
        buf = bufs[b]

        @plsc.parallel_loop(0, _L, unroll=2)
        def _add(r):
            for jj in range(_EMBED // 16):
                sl = pl.ds(16 * jj, 16)
                pv = pos_v[r, sl]
                buf[r, sl] = buf[r, sl] + pv
                r2 = r + _L
                buf[r2, sl] = buf[r2, sl] + pv

        base = wid * _RPW + sc * _SUPER
        pltpu.async_copy(buf, out_hbm.at[pl.ds(base, _SUPER)], wsems[b])

    start(0)
    for sc in range(_NSUPER):
        if sc + 1 < _NSUPER:
            start(sc + 1)
        process(sc)
    pltpu.make_async_copy(out_hbm.at[pl.ds(0, _SUPER)], tok0, wsem0).wait()
    pltpu.make_async_copy(out_hbm.at[pl.ds(0, _SUPER)], tok1, wsem1).wait()


@jax.jit
def _embed(xf, tok_table, pos_table):
    mesh = plsc.VectorSubcoreMesh(core_axis_name="c", subcore_axis_name="s")
    return pl.kernel(
        _embed_kernel,
        out_type=jax.ShapeDtypeStruct((_ROWS, _EMBED), jnp.float32),
        mesh=mesh,
        scratch_types=[
            pltpu.VMEM((_RPW // _G, _G), jnp.int32),
            pltpu.VMEM((_SUPER, _EMBED), jnp.float32),
            pltpu.VMEM((_SUPER, _EMBED), jnp.float32),
            pltpu.VMEM((_L, _EMBED), jnp.float32),
            pltpu.SemaphoreType.DMA,
            pltpu.SemaphoreType.DMA,
            pltpu.SemaphoreType.DMA,
            pltpu.SemaphoreType.DMA,
        ],
        compiler_params=pltpu.CompilerParams(use_tc_tiling_on_sc=False),
    )(xf, tok_table, pos_table)


def kernel(x, tok_table, pos_table):
    xf = jnp.reshape(x, (_ROWS // _G, _G)).astype(jnp.int32)
    out = _embed(xf, tok_table, pos_table)
    return jnp.reshape(out, (_B, _L, _EMBED))


# speedup vs baseline: 1.2318x; 1.0001x over previous
import jax
import jax.numpy as jnp
from jax import lax
from jax.experimental import pallas as pl
from jax.experimental.pallas import tpu as pltpu
from jax.experimental.pallas import tpu_sc as plsc

_VOCAB = 1000000
_EMBED = 64
_B, _L = 1024, 200
_NW = 32
_ROWS = _B * _L
_RPW = _ROWS // _NW
_BPW = _B // _NW
_SUPER = 2 * _L
_NSUPER = _RPW // _SUPER
_G = 100
_NG = _SUPER // _G


def _embed_kernel(x_hbm, tok_hbm, pos_hbm, out_hbm,
                  idx_all, tok0, tok1, pos_v,
                  gsem0, gsem1, wsem0, wsem1):
    c = lax.axis_index("c")
    s = lax.axis_index("s")
    wid = s * 2 + c
    bufs = (tok0, tok1)
    gsems = (gsem0, gsem1)
    wsems = (wsem0, wsem1)

    pltpu.sync_copy(pos_hbm.at[pl.ds(0, _L)], pos_v)
    pltpu.sync_copy(x_hbm.at[pl.ds(wid * (_RPW // _G), _RPW // _G)], idx_all)

    descs = {}

    def start(sc):
        b = sc & 1
        if sc >= 2:
            pltpu.make_async_copy(out_hbm.at[pl.ds(0, 2)],
                                  bufs[b], wsems[b]).wait()
        dlist = []
        for j in range(_NG):
            d = pltpu.async_copy(
                tok_hbm.at[idx_all.at[_NG * sc + j]],
                bufs[b].at[j // 2, pl.ds(_G * (j % 2), _G)],
                gsems[b])
            dlist.append(d)
        descs[sc] = dlist

    def process(sc):
        b = sc & 1
        for d in descs[sc]:
            d.wait()
        buf = bufs[b]

        @plsc.parallel_loop(0, _L, unroll=2)
        def _add(r):
            for jj in range(_EMBED // 16):
                sl = pl.ds(16 * jj, 16)
                pv = pos_v[r, sl]
                buf[0, r, sl] = buf[0, r, sl] + pv
                buf[1, r, sl] = buf[1, r, sl] + pv

        batch = wid * _BPW + 2 * sc
        pltpu.async_copy(buf, out_hbm.at[pl.ds(batch, 2)], wsems[b])

    start(0)
    for sc in range(_NSUPER):
        if sc + 1 < _NSUPER:
            start(sc + 1)
        process(sc)
    pltpu.make_async_copy(out_hbm.at[pl.ds(0, 2)], tok0, wsem0).wait()
    pltpu.make_async_copy(out_hbm.at[pl.ds(0, 2)], tok1, wsem1).wait()


@jax.jit
def _embed(xf, tok_table, pos_table):
    mesh = plsc.VectorSubcoreMesh(core_axis_name="c", subcore_axis_name="s")
    return pl.kernel(
        _embed_kernel,
        out_type=jax.ShapeDtypeStruct((_B, _L, _EMBED), jnp.float32),
        mesh=mesh,
        scratch_types=[
            pltpu.VMEM((_RPW // _G, _G), jnp.int32),
            pltpu.VMEM((2, _L, _EMBED), jnp.float32),
            pltpu.VMEM((2, _L, _EMBED), jnp.float32),
            pltpu.VMEM((_L, _EMBED), jnp.float32),
            pltpu.SemaphoreType.DMA,
            pltpu.SemaphoreType.DMA,
            pltpu.SemaphoreType.DMA,
            pltpu.SemaphoreType.DMA,
        ],
        compiler_params=pltpu.CompilerParams(use_tc_tiling_on_sc=False),
    )(xf, tok_table, pos_table)


def kernel(x, tok_table, pos_table):
    xf = jnp.reshape(x, (_ROWS // _G, _G)).astype(jnp.int32)
    return _embed(xf, tok_table, pos_table)


# speedup vs baseline: 1.2330x; 1.0010x over previous
import jax
import jax.numpy as jnp
from jax import lax
from jax.experimental import pallas as pl
from jax.experimental.pallas import tpu as pltpu
from jax.experimental.pallas import tpu_sc as plsc

_VOCAB = 1000000
_EMBED = 64
_B, _L = 1024, 200
_NW = 32
_ROWS = _B * _L
_RPW = _ROWS // _NW
_BPW = _B // _NW
_SUPER = 2 * _L
_NSUPER = _RPW // _SUPER
_G = 100
_NG = _SUPER // _G


def _embed_kernel(x_hbm, tok_hbm, pos_hbm, out_hbm,
                  idx_all, tok0, tok1, pos_v,
                  gsem0, gsem1, wsem0, wsem1):
    c = lax.axis_index("c")
    s = lax.axis_index("s")
    wid = s * 2 + c
    bufs = (tok0, tok1)
    gsems = (gsem0, gsem1)
    wsems = (wsem0, wsem1)

    pltpu.sync_copy(pos_hbm.at[pl.ds(0, _L)], pos_v)
    pltpu.sync_copy(x_hbm.at[pl.ds(wid * _RPW, _RPW)], idx_all)

    descs = {}

    def start(sc):
        b = sc & 1
        if sc >= 2:
            pltpu.make_async_copy(out_hbm.at[pl.ds(0, 2)],
                                  bufs[b], wsems[b]).wait()
        dlist = []
        for rep in range(2):
            off = sc * _SUPER + rep * _L
            dlist.append(pltpu.async_copy(
                tok_hbm.at[idx_all.at[pl.ds(off, 128)]],
                bufs[b].at[rep, pl.ds(0, 128)],
                gsems[b]))
            dlist.append(pltpu.async_copy(
                tok_hbm.at[idx_all.at[pl.ds(off + 128, 72)]],
                bufs[b].at[rep, pl.ds(128, 72)],
                gsems[b]))
        descs[sc] = dlist

    def process(sc):
        b = sc & 1
        for d in descs[sc]:
            d.wait()
        buf = bufs[b]

        @plsc.parallel_loop(0, _L, unroll=2)
        def _add(r):
            for jj in range(_EMBED // 16):
                sl = pl.ds(16 * jj, 16)
                pv = pos_v[r, sl]
                buf[0, r, sl] = buf[0, r, sl] + pv
                buf[1, r, sl] = buf[1, r, sl] + pv

        batch = wid * _BPW + 2 * sc
        pltpu.async_copy(buf, out_hbm.at[pl.ds(batch, 2)], wsems[b])

    start(0)
    for sc in range(_NSUPER):
        if sc + 1 < _NSUPER:
            start(sc + 1)
        process(sc)
    pltpu.make_async_copy(out_hbm.at[pl.ds(0, 2)], tok0, wsem0).wait()
    pltpu.make_async_copy(out_hbm.at[pl.ds(0, 2)], tok1, wsem1).wait()


@jax.jit
def _embed(xf, tok_table, pos_table):
    mesh = plsc.VectorSubcoreMesh(core_axis_name="c", subcore_axis_name="s")
    return pl.kernel(
        _embed_kernel,
        out_type=jax.ShapeDtypeStruct((_B, _L, _EMBED), jnp.float32),
        mesh=mesh,
        scratch_types=[
            pltpu.VMEM((_RPW,), jnp.int32),
            pltpu.VMEM((2, _L, _EMBED), jnp.float32),
            pltpu.VMEM((2, _L, _EMBED), jnp.float32),
            pltpu.VMEM((_L, _EMBED), jnp.float32),
            pltpu.SemaphoreType.DMA,
            pltpu.SemaphoreType.DMA,
            pltpu.SemaphoreType.DMA,
            pltpu.SemaphoreType.DMA,
        ],
        compiler_params=pltpu.CompilerParams(use_tc_tiling_on_sc=False),
    )(xf, tok_table, pos_table)


def kernel(x, tok_table, pos_table):
    xf = jnp.reshape(x, (_ROWS,)).astype(jnp.int32)
    return _embed(xf, tok_table, pos_table)
